# baseline (speedup 1.0000x reference)
import functools

import jax
import jax.numpy as jnp
from jax import lax
from jax.experimental import pallas as pl
from jax.experimental.pallas import tpu as pltpu

_NUM_EXPS = [8, 16, 64, 256, 1024]
_KS = [4, 4, 32, 32, 256]
_OFFS = [0, 8, 24, 88, 344, 1368]
_HID = 256
_OUT = 3
_NB = 2
_NC = 256
_GIN = 3072
_GOUT = 1368
_GATE_TILE = 128
_GATE_STEPS = 11


def _lower_tri(ne):
    row = lax.broadcasted_iota(jnp.int32, (ne, ne), 0)
    col = lax.broadcasted_iota(jnp.int32, (ne, ne), 1)
    return (row <= col).astype(jnp.float32)


def _topk_mask(g, k):
    ne = g.shape[1]
    bits = lax.bitcast_convert_type(jnp.abs(g), jnp.int32)

    def bs_body(_, carry):
        lo, hi = carry
        mid = lo + (hi - lo + 1) // 2
        cnt = jnp.sum((bits >= mid).astype(jnp.int32), axis=1, keepdims=True)
        ok = cnt >= k
        return jnp.where(ok, mid, lo), jnp.where(ok, hi, mid - 1)

    lo0 = jnp.zeros((_NB, 1), jnp.int32)
    hi0 = jnp.full((_NB, 1), 0x7F800000, jnp.int32)
    t, _ = lax.fori_loop(0, 32, bs_body, (lo0, hi0))

    gt = bits > t
    eq = bits == t
    c1 = jnp.sum(gt.astype(jnp.int32), axis=1, keepdims=True)
    rank = jnp.dot(eq.astype(jnp.float32), _lower_tri(ne),
                   preferred_element_type=jnp.float32)
    return gt | (eq & (rank <= (k - c1).astype(jnp.float32)))


def _col_to_row(vcol, k):
    r = lax.broadcasted_iota(jnp.int32, (k, k), 0)
    c = lax.broadcasted_iota(jnp.int32, (k, k), 1)
    return jnp.sum(jnp.where(r == c, jnp.broadcast_to(vcol, (k, k)), 0.0),
                   axis=0, keepdims=True)


def _gate_body(topk_ref, x_ref, wg_ref, bg_ref,
               out_ref, i1_ref, s1_ref, i2_ref, s2_ref, i3_ref, s3_ref,
               raw_ref):
    j = pl.program_id(0)
    raw_ref[:, pl.ds(j * _GATE_TILE, _GATE_TILE)] = jnp.dot(
        x_ref[...], wg_ref[...], preferred_element_type=jnp.float32)

    @pl.when(j == _GATE_STEPS - 1)
    def _epilogue():
        use_topk = topk_ref[0, 0] != 0
        idx_refs = {1: i1_ref, 2: i2_ref, 3: i3_ref}
        gs_refs = {1: s1_ref, 2: s2_ref, 3: s3_ref}
        for li in range(5):
            off, ne, k = _OFFS[li], _NUM_EXPS[li], _KS[li]
            g = raw_ref[:, off:off + ne] + bg_ref[:, off:off + ne]
            mask = _topk_mask(g, k)
            z = jnp.where(mask, g, 0.0)
            nrm = jnp.sqrt(jnp.sum(g * g, axis=1, keepdims=True))
            gn = g / jnp.maximum(nrm, 1e-12)
            gates = jnp.where(use_topk, z, gn)
            out_ref[:, off:off + ne] = gates
            if li not in idx_refs:
                continue
            ranksel = jnp.dot(mask.astype(jnp.float32), _lower_tri(ne),
                              preferred_element_type=jnp.float32)
            srow = lax.broadcasted_iota(jnp.int32, (k, ne), 0)
            jrow = lax.broadcasted_iota(jnp.int32, (k, ne),
                                        1).astype(jnp.float32)
            for b in range(_NB):
                rt = (jnp.broadcast_to(ranksel[b:b + 1, :], (k, ne))
                      == (srow + 1).astype(jnp.float32))
                rt = rt & jnp.broadcast_to(mask[b:b + 1, :], (k, ne))
                sel_col = jnp.sum(jnp.where(rt, jrow, 0.0),
                                  axis=1, keepdims=True)
                gs_col = jnp.sum(
                    jnp.where(rt, jnp.broadcast_to(gates[b:b + 1, :],
                                                   (k, ne)), 0.0),
                    axis=1, keepdims=True)
                idx_refs[li][b:b + 1, :] = _col_to_row(sel_col,
                                                       k).astype(jnp.int32)
                gs_refs[li][b:b + 1, :] = _col_to_row(gs_col, k)


def _compute_gates(x_img, Wg, bg, topk_s):
    outs = [jax.ShapeDtypeStruct((_NB, _GOUT), jnp.float32)]
    for li in (1, 2, 3):
        k = _KS[li]
        outs.append(jax.ShapeDtypeStruct((_NB, k), jnp.int32))
        outs.append(jax.ShapeDtypeStruct((_NB, k), jnp.float32))
    full = lambda j: tuple(0 for _ in range(2))
    return pl.pallas_call(
        _gate_body,
        grid=(_GATE_STEPS,),
        in_specs=[
            pl.BlockSpec(memory_space=pltpu.SMEM),
            pl.BlockSpec((_NB, _GIN), lambda j: (0, 0)),
            pl.BlockSpec((_GIN, _GATE_TILE), lambda j: (0, j)),
            pl.BlockSpec((1, _GOUT), lambda j: (0, 0)),
        ],
        out_specs=[
            pl.BlockSpec((_NB, _GOUT), full),
            pl.BlockSpec((_NB, _KS[1]), full), pl.BlockSpec((_NB, _KS[1]), full),
            pl.BlockSpec((_NB, _KS[2]), full), pl.BlockSpec((_NB, _KS[2]), full),
            pl.BlockSpec((_NB, _KS[3]), full), pl.BlockSpec((_NB, _KS[3]), full),
        ],
        out_shape=outs,
        scratch_shapes=[pltpu.VMEM((_NB, _GATE_TILE * _GATE_STEPS),
                                   jnp.float32)],
    )(topk_s, x_img, Wg, bg)


def _gate_col(g_row, tile, ne):
    jj = lax.broadcasted_iota(jnp.int32, (tile, ne), 0)
    ee = lax.broadcasted_iota(jnp.int32, (tile, ne), 1)
    sel = (jj % ne) == ee
    return jnp.sum(jnp.where(sel, jnp.broadcast_to(g_row, (tile, ne)), 0.0),
                   axis=1, keepdims=True)


def _combine_matmul(yb, g_row, ne, h):
    tile = h * ne
    gcol = _gate_col(g_row, tile, ne)
    jj = lax.broadcasted_iota(jnp.int32, (tile, h), 0)
    dd = lax.broadcasted_iota(jnp.int32, (tile, h), 1)
    gmat = jnp.where(jj // ne == dd, jnp.broadcast_to(gcol, (tile, h)), 0.0)
    return jnp.dot(yb, gmat, preferred_element_type=jnp.float32)


def _l0_body(coords_ref, w_ref, b_ref, g_ref, out_ref):
    y = jnp.sin(30.0 * (jnp.dot(coords_ref[...], w_ref[...],
                                preferred_element_type=jnp.float32)
                        + b_ref[...]))
    for b in range(_NB):
        out_ref[pl.ds(b * _NC, _NC), :] = _combine_matmul(
            y, g_ref[b:b + 1, :], _NUM_EXPS[0], _HID)


def _layer0(coords, W0, b0, g0):
    return pl.pallas_call(
        _l0_body,
        out_shape=jax.ShapeDtypeStruct((_NB * _NC, _HID), jnp.float32),
    )(coords, W0, b0, g0)


def _mid_body(x_ref, w_ref, b_ref, idx_ref, gs_ref, out_ref, *, ne, k, tile,
              dense=False):
    h = tile // ne
    u = jnp.dot(x_ref[...], w_ref[...],
                preferred_element_type=jnp.float32) + b_ref[...]
    if dense:
        y = jnp.sin(30.0 * u)
        for b in range(_NB):
            out_ref[0, pl.ds(b * _NC, _NC), :] = _combine_matmul(
                y[b * _NC:(b + 1) * _NC, :], gs_ref[b:b + 1, :], ne, h)
        return
    for b in range(_NB):
        u3 = u[b * _NC:(b + 1) * _NC, :].reshape(_NC, h, ne)
        idx3 = jnp.broadcast_to(idx_ref[b:b + 1, :].reshape(1, 1, k),
                                (_NC, h, k))
        if ne <= 128:
            us = jnp.take_along_axis(u3, idx3, axis=2)
        else:
            p0 = jnp.take_along_axis(u3[:, :, :128],
                                     jnp.minimum(idx3, 127), axis=2)
            p1 = jnp.take_along_axis(u3[:, :, 128:],
                                     jnp.maximum(idx3 - 128, 0), axis=2)
            us = jnp.where(idx3 < 128, p0, p1)
        y = jnp.sin(30.0 * us)
        yb = y.astype(jnp.bfloat16).astype(jnp.float32)
        gb = gs_ref[b:b + 1, :].reshape(1, 1, k)
        gb = gb.astype(jnp.bfloat16).astype(jnp.float32)
        out_ref[0, pl.ds(b * _NC, _NC), :] = jnp.sum(yb * gb, axis=2)


def _mid_layer(x, W, b, idx, gs, *, ne, k, d, tile, dense=False):
    ncols = W.shape[1]
    steps = ncols // tile
    h = tile // ne
    gs_cols = ne if dense else k
    body = functools.partial(_mid_body, ne=ne, k=k, tile=tile, dense=dense)
    out3 = pl.pallas_call(
        body,
        grid=(steps,),
        in_specs=[
            pl.BlockSpec((_NB * _NC, _HID), lambda j: (0, 0)),
            pl.BlockSpec((_HID, tile), lambda j: (0, j)),
            pl.BlockSpec((1, tile), lambda j: (0, j)),
            pl.BlockSpec((_NB, k), lambda j: (0, 0)),
            pl.BlockSpec((_NB, gs_cols), lambda j: (0, 0)),
        ],
        out_specs=pl.BlockSpec((1, _NB * _NC, h), lambda j: (j, 0, 0)),
        out_shape=jax.ShapeDtypeStruct((steps, _NB * _NC, h), jnp.float32),
    )(x, W, b, idx, gs)
    return out3.transpose(1, 0, 2).reshape(_NB * _NC, d)


def _l4_body(x_ref, w_ref, b_ref, g_ref, out_ref):
    ne = _NUM_EXPS[4]
    u = jnp.dot(x_ref[...], w_ref[...],
                preferred_element_type=jnp.float32) + b_ref[...]
    for b in range(_NB):
        out_ref[b, :, :] = _combine_matmul(
            u[b * _NC:(b + 1) * _NC, :], g_ref[b:b + 1, :], ne, _OUT)


def _layer4(x, W4, b4, g4):
    return pl.pallas_call(
        _l4_body,
        out_shape=jax.ShapeDtypeStruct((_NB, _NC, _OUT), jnp.float32),
    )(x, W4, b4, g4)


def kernel(img, coords, Wg, bg, W0, b0, W1, b1, W2, b2, W3, b3, W4, b4,
           top_k):
    x_img = img.reshape(_NB, _GIN)
    topk_s = jnp.asarray(top_k, jnp.int32).reshape(1, 1)
    gates, i1, s1, i2, s2, i3, s3 = _compute_gates(
        x_img, Wg, bg.reshape(1, _GOUT), topk_s)

    g0 = gates[:, _OFFS[0]:_OFFS[1]]
    g4 = gates[:, _OFFS[4]:_OFFS[5]]

    x = _layer0(coords, W0, b0.reshape(1, -1), g0)
    g1 = gates[:, _OFFS[1]:_OFFS[2]]
    x = _mid_layer(x, W1, b1.reshape(1, -1), i1, g1,
                   ne=16, k=4, d=_HID, tile=2048, dense=True)
    x = _mid_layer(x, W2, b2.reshape(1, -1), i2, s2,
                   ne=64, k=32, d=_HID, tile=2048)
    x = _mid_layer(x, W3, b3.reshape(1, -1), i3, s3,
                   ne=256, k=32, d=_HID, tile=2048)
    return _layer4(x, W4, b4.reshape(1, -1), g4)


# speedup vs baseline: 1.9521x; 1.9521x over previous
import functools

import jax
import jax.numpy as jnp
from jax import lax
from jax.experimental import pallas as pl
from jax.experimental.pallas import tpu as pltpu

_NUM_EXPS = [8, 16, 64, 256, 1024]
_KS = [4, 4, 32, 32, 256]
_OFFS = [0, 8, 24, 88, 344, 1368]
_HID = 256
_OUT = 3
_NB = 2
_NC = 256
_GIN = 3072
_GOUT = 1368
_GATE_TILE = 128
_GATE_STEPS = 11


def _lower_tri(ne):
    row = lax.broadcasted_iota(jnp.int32, (ne, ne), 0)
    col = lax.broadcasted_iota(jnp.int32, (ne, ne), 1)
    return (row <= col).astype(jnp.float32)


def _topk_mask(g, k):
    ne = g.shape[1]
    bits = lax.bitcast_convert_type(jnp.abs(g), jnp.int32)

    def bs_body(_, carry):
        lo, hi = carry
        mid = lo + (hi - lo + 1) // 2
        cnt = jnp.sum((bits >= mid).astype(jnp.int32), axis=1, keepdims=True)
        ok = cnt >= k
        return jnp.where(ok, mid, lo), jnp.where(ok, hi, mid - 1)

    lo0 = jnp.zeros((_NB, 1), jnp.int32)
    hi0 = jnp.full((_NB, 1), 0x7F800000, jnp.int32)
    t, _ = lax.fori_loop(0, 32, bs_body, (lo0, hi0))

    gt = bits > t
    eq = bits == t
    c1 = jnp.sum(gt.astype(jnp.int32), axis=1, keepdims=True)
    rank = jnp.dot(eq.astype(jnp.float32), _lower_tri(ne),
                   preferred_element_type=jnp.float32)
    return gt | (eq & (rank <= (k - c1).astype(jnp.float32)))


def _col_to_row(vcol, k):
    r = lax.broadcasted_iota(jnp.int32, (k, k), 0)
    c = lax.broadcasted_iota(jnp.int32, (k, k), 1)
    return jnp.sum(jnp.where(r == c, jnp.broadcast_to(vcol, (k, k)), 0.0),
                   axis=0, keepdims=True)


def _gate_body(topk_ref, x_ref, wg_ref, bg_ref,
               out_ref, i1_ref, s1_ref, i2_ref, s2_ref, i3_ref, s3_ref,
               raw_ref):
    j = pl.program_id(0)
    raw_ref[:, pl.ds(j * _GATE_TILE, _GATE_TILE)] = jnp.dot(
        x_ref[...], wg_ref[...], preferred_element_type=jnp.float32)

    @pl.when(j == _GATE_STEPS - 1)
    def _epilogue():
        use_topk = topk_ref[0, 0] != 0
        idx_refs = {1: i1_ref, 2: i2_ref, 3: i3_ref}
        gs_refs = {1: s1_ref, 2: s2_ref, 3: s3_ref}
        for li in range(5):
            off, ne, k = _OFFS[li], _NUM_EXPS[li], _KS[li]
            g = raw_ref[:, off:off + ne] + bg_ref[:, off:off + ne]
            mask = _topk_mask(g, k)
            z = jnp.where(mask, g, 0.0)
            nrm = jnp.sqrt(jnp.sum(g * g, axis=1, keepdims=True))
            gn = g / jnp.maximum(nrm, 1e-12)
            gates = jnp.where(use_topk, z, gn)
            out_ref[:, off:off + ne] = gates
            if li not in idx_refs:
                continue
            ranksel = jnp.dot(mask.astype(jnp.float32), _lower_tri(ne),
                              preferred_element_type=jnp.float32)
            srow = lax.broadcasted_iota(jnp.int32, (k, ne), 0)
            jrow = lax.broadcasted_iota(jnp.int32, (k, ne),
                                        1).astype(jnp.float32)
            for b in range(_NB):
                rt = (jnp.broadcast_to(ranksel[b:b + 1, :], (k, ne))
                      == (srow + 1).astype(jnp.float32))
                rt = rt & jnp.broadcast_to(mask[b:b + 1, :], (k, ne))
                sel_col = jnp.sum(jnp.where(rt, jrow, 0.0),
                                  axis=1, keepdims=True)
                gs_col = jnp.sum(
                    jnp.where(rt, jnp.broadcast_to(gates[b:b + 1, :],
                                                   (k, ne)), 0.0),
                    axis=1, keepdims=True)
                idx_refs[li][b:b + 1, :] = _col_to_row(sel_col,
                                                       k).astype(jnp.int32)
                gs_refs[li][b:b + 1, :] = _col_to_row(gs_col, k)


def _compute_gates(x_img, Wg, bg, topk_s):
    outs = [jax.ShapeDtypeStruct((_NB, _GOUT), jnp.float32)]
    for li in (1, 2, 3):
        k = _KS[li]
        outs.append(jax.ShapeDtypeStruct((_NB, k), jnp.int32))
        outs.append(jax.ShapeDtypeStruct((_NB, k), jnp.float32))
    full = lambda j: tuple(0 for _ in range(2))
    return pl.pallas_call(
        _gate_body,
        grid=(_GATE_STEPS,),
        in_specs=[
            pl.BlockSpec(memory_space=pltpu.SMEM),
            pl.BlockSpec((_NB, _GIN), lambda j: (0, 0)),
            pl.BlockSpec((_GIN, _GATE_TILE), lambda j: (0, j)),
            pl.BlockSpec((1, _GOUT), lambda j: (0, 0)),
        ],
        out_specs=[
            pl.BlockSpec((_NB, _GOUT), full),
            pl.BlockSpec((_NB, _KS[1]), full), pl.BlockSpec((_NB, _KS[1]), full),
            pl.BlockSpec((_NB, _KS[2]), full), pl.BlockSpec((_NB, _KS[2]), full),
            pl.BlockSpec((_NB, _KS[3]), full), pl.BlockSpec((_NB, _KS[3]), full),
        ],
        out_shape=outs,
        scratch_shapes=[pltpu.VMEM((_NB, _GATE_TILE * _GATE_STEPS),
                                   jnp.float32)],
    )(topk_s, x_img, Wg, bg)


def _gate_col(g_row, tile, ne):
    jj = lax.broadcasted_iota(jnp.int32, (tile, ne), 0)
    ee = lax.broadcasted_iota(jnp.int32, (tile, ne), 1)
    sel = (jj % ne) == ee
    return jnp.sum(jnp.where(sel, jnp.broadcast_to(g_row, (tile, ne)), 0.0),
                   axis=1, keepdims=True)


def _combine_matmul(yb, g_row, ne, h):
    tile = h * ne
    gcol = _gate_col(g_row, tile, ne)
    jj = lax.broadcasted_iota(jnp.int32, (tile, h), 0)
    dd = lax.broadcasted_iota(jnp.int32, (tile, h), 1)
    gmat = jnp.where(jj // ne == dd, jnp.broadcast_to(gcol, (tile, h)), 0.0)
    return jnp.dot(yb, gmat, preferred_element_type=jnp.float32)


def _l0_body(coords_ref, w_ref, b_ref, g_ref, out_ref):
    y = jnp.sin(30.0 * (jnp.dot(coords_ref[...], w_ref[...],
                                preferred_element_type=jnp.float32)
                        + b_ref[...]))
    for b in range(_NB):
        out_ref[pl.ds(b * _NC, _NC), :] = _combine_matmul(
            y, g_ref[b:b + 1, :], _NUM_EXPS[0], _HID)


def _layer0(coords, W0, b0, g0):
    return pl.pallas_call(
        _l0_body,
        out_shape=jax.ShapeDtypeStruct((_NB * _NC, _HID), jnp.float32),
    )(coords, W0, b0, g0)


def _mid_body(x_ref, w_ref, b_ref, idx_ref, gs_ref, out_ref, *, ne, k, tile,
              dense=False):
    h = tile // ne
    u = jnp.dot(x_ref[...], w_ref[...],
                preferred_element_type=jnp.float32) + b_ref[...]
    if dense:
        y = jnp.sin(30.0 * u)
        for b in range(_NB):
            out_ref[0, pl.ds(b * _NC, _NC), :] = _combine_matmul(
                y[b * _NC:(b + 1) * _NC, :], gs_ref[b:b + 1, :], ne, h)
        return
    for b in range(_NB):
        ub = u[b * _NC:(b + 1) * _NC, :]
        i2 = jnp.broadcast_to(idx_ref[b:b + 1, :], (_NC, k))
        parts = []
        for dd in range(h):
            seg = ub[:, dd * ne:(dd + 1) * ne]
            if ne <= 128:
                parts.append(jnp.take_along_axis(seg, i2, axis=1))
            else:
                p0 = jnp.take_along_axis(seg[:, :128],
                                         jnp.minimum(i2, 127), axis=1)
                p1 = jnp.take_along_axis(seg[:, 128:],
                                         jnp.maximum(i2 - 128, 0), axis=1)
                parts.append(jnp.where(i2 < 128, p0, p1))
        ysel = jnp.sin(30.0 * jnp.concatenate(parts, axis=1))
        out_ref[0, pl.ds(b * _NC, _NC), :] = _combine_matmul(
            ysel, gs_ref[b:b + 1, :], k, h)


def _mid_layer(x, W, b, idx, gs, *, ne, k, d, tile, dense=False):
    ncols = W.shape[1]
    steps = ncols // tile
    h = tile // ne
    gs_cols = ne if dense else k
    body = functools.partial(_mid_body, ne=ne, k=k, tile=tile, dense=dense)
    out3 = pl.pallas_call(
        body,
        grid=(steps,),
        in_specs=[
            pl.BlockSpec((_NB * _NC, _HID), lambda j: (0, 0)),
            pl.BlockSpec((_HID, tile), lambda j: (0, j)),
            pl.BlockSpec((1, tile), lambda j: (0, j)),
            pl.BlockSpec((_NB, k), lambda j: (0, 0)),
            pl.BlockSpec((_NB, gs_cols), lambda j: (0, 0)),
        ],
        out_specs=pl.BlockSpec((1, _NB * _NC, h), lambda j: (j, 0, 0)),
        out_shape=jax.ShapeDtypeStruct((steps, _NB * _NC, h), jnp.float32),
    )(x, W, b, idx, gs)
    return out3.transpose(1, 0, 2).reshape(_NB * _NC, d)


def _l4_body(x_ref, w_ref, b_ref, g_ref, out_ref):
    ne = _NUM_EXPS[4]
    u = jnp.dot(x_ref[...], w_ref[...],
                preferred_element_type=jnp.float32) + b_ref[...]
    for b in range(_NB):
        out_ref[b, :, :] = _combine_matmul(
            u[b * _NC:(b + 1) * _NC, :], g_ref[b:b + 1, :], ne, _OUT)


def _layer4(x, W4, b4, g4):
    return pl.pallas_call(
        _l4_body,
        out_shape=jax.ShapeDtypeStruct((_NB, _NC, _OUT), jnp.float32),
    )(x, W4, b4, g4)


def kernel(img, coords, Wg, bg, W0, b0, W1, b1, W2, b2, W3, b3, W4, b4,
           top_k):
    x_img = img.reshape(_NB, _GIN)
    topk_s = jnp.asarray(top_k, jnp.int32).reshape(1, 1)
    gates, i1, s1, i2, s2, i3, s3 = _compute_gates(
        x_img, Wg, bg.reshape(1, _GOUT), topk_s)

    g0 = gates[:, _OFFS[0]:_OFFS[1]]
    g4 = gates[:, _OFFS[4]:_OFFS[5]]

    x = _layer0(coords, W0, b0.reshape(1, -1), g0)
    g1 = gates[:, _OFFS[1]:_OFFS[2]]
    x = _mid_layer(x, W1, b1.reshape(1, -1), i1, g1,
                   ne=16, k=4, d=_HID, tile=2048, dense=True)
    g2 = gates[:, _OFFS[2]:_OFFS[3]]
    x = _mid_layer(x, W2, b2.reshape(1, -1), i2, g2,
                   ne=64, k=32, d=_HID, tile=2048, dense=True)
    x = _mid_layer(x, W3, b3.reshape(1, -1), i3, s3,
                   ne=256, k=32, d=_HID, tile=2048)
    return _layer4(x, W4, b4.reshape(1, -1), g4)


# speedup vs baseline: 2.0474x; 1.0488x over previous
import functools

import jax
import jax.numpy as jnp
from jax import lax
from jax.experimental import pallas as pl
from jax.experimental.pallas import tpu as pltpu

_NUM_EXPS = [8, 16, 64, 256, 1024]
_KS = [4, 4, 32, 32, 256]
_OFFS = [0, 8, 24, 88, 344, 1368]
_HID = 256
_OUT = 3
_NB = 2
_NC = 256
_GIN = 3072
_GOUT = 1368
_GATE_TILE = 128
_GATE_STEPS = 11


def _lower_tri(ne):
    row = lax.broadcasted_iota(jnp.int32, (ne, ne), 0)
    col = lax.broadcasted_iota(jnp.int32, (ne, ne), 1)
    return (row <= col).astype(jnp.float32)


def _topk_mask(g, k):
    ne = g.shape[1]
    bits = lax.bitcast_convert_type(jnp.abs(g), jnp.int32)

    def bs_body(_, carry):
        lo, hi = carry
        mid = lo + (hi - lo + 1) // 2
        cnt = jnp.sum((bits >= mid).astype(jnp.int32), axis=1, keepdims=True)
        ok = cnt >= k
        return jnp.where(ok, mid, lo), jnp.where(ok, hi, mid - 1)

    lo0 = jnp.zeros((_NB, 1), jnp.int32)
    hi0 = jnp.full((_NB, 1), 0x7F800000, jnp.int32)
    t, _ = lax.fori_loop(0, 32, bs_body, (lo0, hi0))

    gt = bits > t
    eq = bits == t
    c1 = jnp.sum(gt.astype(jnp.int32), axis=1, keepdims=True)
    rank = jnp.dot(eq.astype(jnp.float32), _lower_tri(ne),
                   preferred_element_type=jnp.float32)
    return gt | (eq & (rank <= (k - c1).astype(jnp.float32)))


def _col_to_row(vcol, k):
    r = lax.broadcasted_iota(jnp.int32, (k, k), 0)
    c = lax.broadcasted_iota(jnp.int32, (k, k), 1)
    return jnp.sum(jnp.where(r == c, jnp.broadcast_to(vcol, (k, k)), 0.0),
                   axis=0, keepdims=True)


def _gate_body(topk_ref, x_ref, wg_ref, bg_ref,
               out_ref, i1_ref, s1_ref, i2_ref, s2_ref, i3_ref, s3_ref,
               raw_ref):
    j = pl.program_id(0)
    raw_ref[:, pl.ds(j * _GATE_TILE, _GATE_TILE)] = jnp.dot(
        x_ref[...], wg_ref[...], preferred_element_type=jnp.float32)

    @pl.when(j == _GATE_STEPS - 1)
    def _epilogue():
        use_topk = topk_ref[0, 0] != 0
        idx_refs = {1: i1_ref, 2: i2_ref, 3: i3_ref}
        gs_refs = {1: s1_ref, 2: s2_ref, 3: s3_ref}
        for li in range(5):
            off, ne, k = _OFFS[li], _NUM_EXPS[li], _KS[li]
            g = raw_ref[:, off:off + ne] + bg_ref[:, off:off + ne]
            mask = _topk_mask(g, k)
            z = jnp.where(mask, g, 0.0)
            nrm = jnp.sqrt(jnp.sum(g * g, axis=1, keepdims=True))
            gn = g / jnp.maximum(nrm, 1e-12)
            gates = jnp.where(use_topk, z, gn)
            out_ref[:, off:off + ne] = gates
            if li not in idx_refs:
                continue
            ranksel = jnp.dot(mask.astype(jnp.float32), _lower_tri(ne),
                              preferred_element_type=jnp.float32)
            srow = lax.broadcasted_iota(jnp.int32, (k, ne), 0)
            jrow = lax.broadcasted_iota(jnp.int32, (k, ne),
                                        1).astype(jnp.float32)
            for b in range(_NB):
                rt = (jnp.broadcast_to(ranksel[b:b + 1, :], (k, ne))
                      == (srow + 1).astype(jnp.float32))
                rt = rt & jnp.broadcast_to(mask[b:b + 1, :], (k, ne))
                sel_col = jnp.sum(jnp.where(rt, jrow, 0.0),
                                  axis=1, keepdims=True)
                gs_col = jnp.sum(
                    jnp.where(rt, jnp.broadcast_to(gates[b:b + 1, :],
                                                   (k, ne)), 0.0),
                    axis=1, keepdims=True)
                idx_refs[li][b:b + 1, :] = _col_to_row(sel_col,
                                                       k).astype(jnp.int32)
                gs_refs[li][b:b + 1, :] = _col_to_row(gs_col, k)


def _compute_gates(x_img, Wg, bg, topk_s):
    outs = [jax.ShapeDtypeStruct((_NB, _GOUT), jnp.float32)]
    for li in (1, 2, 3):
        k = _KS[li]
        outs.append(jax.ShapeDtypeStruct((_NB, k), jnp.int32))
        outs.append(jax.ShapeDtypeStruct((_NB, k), jnp.float32))
    full = lambda j: tuple(0 for _ in range(2))
    return pl.pallas_call(
        _gate_body,
        grid=(_GATE_STEPS,),
        in_specs=[
            pl.BlockSpec(memory_space=pltpu.SMEM),
            pl.BlockSpec((_NB, _GIN), lambda j: (0, 0)),
            pl.BlockSpec((_GIN, _GATE_TILE), lambda j: (0, j)),
            pl.BlockSpec((1, _GOUT), lambda j: (0, 0)),
        ],
        out_specs=[
            pl.BlockSpec((_NB, _GOUT), full),
            pl.BlockSpec((_NB, _KS[1]), full), pl.BlockSpec((_NB, _KS[1]), full),
            pl.BlockSpec((_NB, _KS[2]), full), pl.BlockSpec((_NB, _KS[2]), full),
            pl.BlockSpec((_NB, _KS[3]), full), pl.BlockSpec((_NB, _KS[3]), full),
        ],
        out_shape=outs,
        scratch_shapes=[pltpu.VMEM((_NB, _GATE_TILE * _GATE_STEPS),
                                   jnp.float32)],
    )(topk_s, x_img, Wg, bg)


def _gate_col(g_row, tile, ne):
    jj = lax.broadcasted_iota(jnp.int32, (tile, ne), 0)
    ee = lax.broadcasted_iota(jnp.int32, (tile, ne), 1)
    sel = (jj % ne) == ee
    return jnp.sum(jnp.where(sel, jnp.broadcast_to(g_row, (tile, ne)), 0.0),
                   axis=1, keepdims=True)


def _combine_matmul(yb, g_row, ne, h):
    tile = h * ne
    gcol = _gate_col(g_row, tile, ne)
    jj = lax.broadcasted_iota(jnp.int32, (tile, h), 0)
    dd = lax.broadcasted_iota(jnp.int32, (tile, h), 1)
    gmat = jnp.where(jj // ne == dd, jnp.broadcast_to(gcol, (tile, h)), 0.0)
    return jnp.dot(yb, gmat, preferred_element_type=jnp.float32)


def _l0_body(coords_ref, w_ref, b_ref, g_ref, out_ref):
    y = jnp.sin(30.0 * (jnp.dot(coords_ref[...], w_ref[...],
                                preferred_element_type=jnp.float32)
                        + b_ref[...]))
    for b in range(_NB):
        out_ref[pl.ds(b * _NC, _NC), :] = _combine_matmul(
            y, g_ref[b:b + 1, :], _NUM_EXPS[0], _HID)


def _layer0(coords, W0, b0, g0):
    return pl.pallas_call(
        _l0_body,
        out_shape=jax.ShapeDtypeStruct((_NB * _NC, _HID), jnp.float32),
    )(coords, W0, b0, g0)


def _mid_body(x_ref, w_ref, b_ref, idx_ref, gs_ref, out_ref, *, ne, k, tile,
              dense=False):
    h = tile // ne
    u = jnp.dot(x_ref[...], w_ref[...],
                preferred_element_type=jnp.float32) + b_ref[...]
    if dense:
        y = jnp.sin(30.0 * u)
        for b in range(_NB):
            out_ref[0, pl.ds(b * _NC, _NC), :] = _combine_matmul(
                y[b * _NC:(b + 1) * _NC, :], gs_ref[b:b + 1, :], ne, h)
        return
    for b in range(_NB):
        ub = u[b * _NC:(b + 1) * _NC, :]
        i2 = jnp.broadcast_to(idx_ref[b:b + 1, :], (_NC, k))
        parts = []
        for dd in range(h):
            seg = ub[:, dd * ne:(dd + 1) * ne]
            if ne <= 128:
                parts.append(jnp.take_along_axis(seg, i2, axis=1))
            else:
                p0 = jnp.take_along_axis(seg[:, :128],
                                         jnp.minimum(i2, 127), axis=1)
                p1 = jnp.take_along_axis(seg[:, 128:],
                                         jnp.maximum(i2 - 128, 0), axis=1)
                parts.append(jnp.where(i2 < 128, p0, p1))
        ysel = jnp.sin(30.0 * jnp.concatenate(parts, axis=1))
        out_ref[0, pl.ds(b * _NC, _NC), :] = _combine_matmul(
            ysel, gs_ref[b:b + 1, :], k, h)


def _mid_layer(x, W, b, idx, gs, *, ne, k, d, tile, dense=False):
    ncols = W.shape[1]
    steps = ncols // tile
    h = tile // ne
    gs_cols = ne if dense else k
    body = functools.partial(_mid_body, ne=ne, k=k, tile=tile, dense=dense)
    out3 = pl.pallas_call(
        body,
        grid=(steps,),
        in_specs=[
            pl.BlockSpec((_NB * _NC, _HID), lambda j: (0, 0)),
            pl.BlockSpec((_HID, tile), lambda j: (0, j)),
            pl.BlockSpec((1, tile), lambda j: (0, j)),
            pl.BlockSpec((_NB, k), lambda j: (0, 0)),
            pl.BlockSpec((_NB, gs_cols), lambda j: (0, 0)),
        ],
        out_specs=pl.BlockSpec((1, _NB * _NC, h), lambda j: (j, 0, 0)),
        out_shape=jax.ShapeDtypeStruct((steps, _NB * _NC, h), jnp.float32),
    )(x, W, b, idx, gs)
    return out3.transpose(1, 0, 2).reshape(_NB * _NC, d)


def _l4_body(x_ref, w_ref, b_ref, g_ref, out_ref):
    ne = _NUM_EXPS[4]
    u = jnp.dot(x_ref[...], w_ref[...],
                preferred_element_type=jnp.float32) + b_ref[...]
    for b in range(_NB):
        out_ref[b, :, :] = _combine_matmul(
            u[b * _NC:(b + 1) * _NC, :], g_ref[b:b + 1, :], ne, _OUT)


def _layer4(x, W4, b4, g4):
    return pl.pallas_call(
        _l4_body,
        out_shape=jax.ShapeDtypeStruct((_NB, _NC, _OUT), jnp.float32),
    )(x, W4, b4, g4)


def kernel(img, coords, Wg, bg, W0, b0, W1, b1, W2, b2, W3, b3, W4, b4,
           top_k):
    x_img = img.reshape(_NB, _GIN)
    topk_s = jnp.asarray(top_k, jnp.int32).reshape(1, 1)
    gates, i1, s1, i2, s2, i3, s3 = _compute_gates(
        x_img, Wg, bg.reshape(1, _GOUT), topk_s)

    g0 = gates[:, _OFFS[0]:_OFFS[1]]
    g4 = gates[:, _OFFS[4]:_OFFS[5]]

    x = _layer0(coords, W0, b0.reshape(1, -1), g0)
    g1 = gates[:, _OFFS[1]:_OFFS[2]]
    x = _mid_layer(x, W1, b1.reshape(1, -1), i1, g1,
                   ne=16, k=4, d=_HID, tile=2048, dense=True)
    x = _mid_layer(x, W2, b2.reshape(1, -1), i2, s2,
                   ne=64, k=32, d=_HID, tile=2048)
    x = _mid_layer(x, W3, b3.reshape(1, -1), i3, s3,
                   ne=256, k=32, d=_HID, tile=2048)
    return _layer4(x, W4, b4.reshape(1, -1), g4)


# speedup vs baseline: 2.1738x; 1.0617x over previous
import functools

import jax
import jax.numpy as jnp
from jax import lax
from jax.experimental import pallas as pl
from jax.experimental.pallas import tpu as pltpu

_NUM_EXPS = [8, 16, 64, 256, 1024]
_KS = [4, 4, 32, 32, 256]
_OFFS = [0, 8, 24, 88, 344, 1368]
_HID = 256
_OUT = 3
_NB = 2
_NC = 256
_GIN = 3072
_GOUT = 1368
_GATE_TILE = 512
_GATE_STEPS = 3


def _lower_tri(ne):
    row = lax.broadcasted_iota(jnp.int32, (ne, ne), 0)
    col = lax.broadcasted_iota(jnp.int32, (ne, ne), 1)
    return (row <= col).astype(jnp.float32)


def _topk_mask(g, k):
    ne = g.shape[1]
    bits = lax.bitcast_convert_type(jnp.abs(g), jnp.int32)

    def bs_body(_, carry):
        lo, hi = carry
        mid = lo + (hi - lo + 1) // 2
        cnt = jnp.sum((bits >= mid).astype(jnp.int32), axis=1, keepdims=True)
        ok = cnt >= k
        return jnp.where(ok, mid, lo), jnp.where(ok, hi, mid - 1)

    lo0 = jnp.zeros((_NB, 1), jnp.int32)
    hi0 = jnp.full((_NB, 1), 0x7F800000, jnp.int32)
    t, _ = lax.fori_loop(0, 32, bs_body, (lo0, hi0))

    gt = bits > t
    eq = bits == t
    c1 = jnp.sum(gt.astype(jnp.int32), axis=1, keepdims=True)
    rank = jnp.dot(eq.astype(jnp.float32), _lower_tri(ne),
                   preferred_element_type=jnp.float32)
    return gt | (eq & (rank <= (k - c1).astype(jnp.float32)))


def _col_to_row(vcol, k):
    r = lax.broadcasted_iota(jnp.int32, (k, k), 0)
    c = lax.broadcasted_iota(jnp.int32, (k, k), 1)
    return jnp.sum(jnp.where(r == c, jnp.broadcast_to(vcol, (k, k)), 0.0),
                   axis=0, keepdims=True)


def _gate_body(topk_ref, x_ref, wg_ref, bg_ref,
               out_ref, i1_ref, s1_ref, i2_ref, s2_ref, i3_ref, s3_ref,
               raw_ref):
    j = pl.program_id(0)
    raw_ref[:, pl.ds(j * _GATE_TILE, _GATE_TILE)] = jnp.dot(
        x_ref[...], wg_ref[...], preferred_element_type=jnp.float32)

    @pl.when(j == _GATE_STEPS - 1)
    def _epilogue():
        use_topk = topk_ref[0, 0] != 0
        idx_refs = {1: i1_ref, 2: i2_ref, 3: i3_ref}
        gs_refs = {1: s1_ref, 2: s2_ref, 3: s3_ref}
        for li in range(5):
            off, ne, k = _OFFS[li], _NUM_EXPS[li], _KS[li]
            g = raw_ref[:, off:off + ne] + bg_ref[:, off:off + ne]
            mask = _topk_mask(g, k)
            z = jnp.where(mask, g, 0.0)
            nrm = jnp.sqrt(jnp.sum(g * g, axis=1, keepdims=True))
            gn = g / jnp.maximum(nrm, 1e-12)
            gates = jnp.where(use_topk, z, gn)
            out_ref[:, off:off + ne] = gates
            if li not in idx_refs:
                continue
            ranksel = jnp.dot(mask.astype(jnp.float32), _lower_tri(ne),
                              preferred_element_type=jnp.float32)
            srow = lax.broadcasted_iota(jnp.int32, (k, ne), 0)
            jrow = lax.broadcasted_iota(jnp.int32, (k, ne),
                                        1).astype(jnp.float32)
            for b in range(_NB):
                rt = (jnp.broadcast_to(ranksel[b:b + 1, :], (k, ne))
                      == (srow + 1).astype(jnp.float32))
                rt = rt & jnp.broadcast_to(mask[b:b + 1, :], (k, ne))
                sel_col = jnp.sum(jnp.where(rt, jrow, 0.0),
                                  axis=1, keepdims=True)
                gs_col = jnp.sum(
                    jnp.where(rt, jnp.broadcast_to(gates[b:b + 1, :],
                                                   (k, ne)), 0.0),
                    axis=1, keepdims=True)
                idx_refs[li][b:b + 1, :] = _col_to_row(sel_col,
                                                       k).astype(jnp.int32)
                gs_refs[li][b:b + 1, :] = _col_to_row(gs_col, k)


def _compute_gates(x_img, Wg, bg, topk_s):
    outs = [jax.ShapeDtypeStruct((_NB, _GOUT), jnp.float32)]
    for li in (1, 2, 3):
        k = _KS[li]
        outs.append(jax.ShapeDtypeStruct((_NB, k), jnp.int32))
        outs.append(jax.ShapeDtypeStruct((_NB, k), jnp.float32))
    full = lambda j: tuple(0 for _ in range(2))
    return pl.pallas_call(
        _gate_body,
        grid=(_GATE_STEPS,),
        in_specs=[
            pl.BlockSpec(memory_space=pltpu.SMEM),
            pl.BlockSpec((_NB, _GIN), lambda j: (0, 0)),
            pl.BlockSpec((_GIN, _GATE_TILE), lambda j: (0, j)),
            pl.BlockSpec((1, _GOUT), lambda j: (0, 0)),
        ],
        out_specs=[
            pl.BlockSpec((_NB, _GOUT), full),
            pl.BlockSpec((_NB, _KS[1]), full), pl.BlockSpec((_NB, _KS[1]), full),
            pl.BlockSpec((_NB, _KS[2]), full), pl.BlockSpec((_NB, _KS[2]), full),
            pl.BlockSpec((_NB, _KS[3]), full), pl.BlockSpec((_NB, _KS[3]), full),
        ],
        out_shape=outs,
        scratch_shapes=[pltpu.VMEM((_NB, _GATE_TILE * _GATE_STEPS),
                                   jnp.float32)],
    )(topk_s, x_img, Wg, bg)


def _gate_col(g_row, tile, ne):
    if ne <= 16:
        r = lax.broadcasted_iota(jnp.int32, (tile, 1), 0) % ne
        gcol = jnp.zeros((tile, 1), jnp.float32)
        for e in range(ne):
            gcol = jnp.where(r == e, g_row[0, e], gcol)
        return gcol
    jj = lax.broadcasted_iota(jnp.int32, (tile, ne), 0)
    ee = lax.broadcasted_iota(jnp.int32, (tile, ne), 1)
    sel = (jj % ne) == ee
    return jnp.sum(jnp.where(sel, jnp.broadcast_to(g_row, (tile, ne)), 0.0),
                   axis=1, keepdims=True)


def _combine_matmul(yb, g_row, ne, h):
    tile = h * ne
    gcol = _gate_col(g_row, tile, ne)
    jj = lax.broadcasted_iota(jnp.int32, (tile, h), 0)
    dd = lax.broadcasted_iota(jnp.int32, (tile, h), 1)
    gmat = jnp.where(jj // ne == dd, jnp.broadcast_to(gcol, (tile, h)), 0.0)
    return jnp.dot(yb, gmat, preferred_element_type=jnp.float32)


def _l0_body(coords_ref, w_ref, b_ref, g_ref, out_ref):
    y = jnp.sin(30.0 * (jnp.dot(coords_ref[...], w_ref[...],
                                preferred_element_type=jnp.float32)
                        + b_ref[...]))
    for b in range(_NB):
        out_ref[pl.ds(b * _NC, _NC), :] = _combine_matmul(
            y, g_ref[b:b + 1, :], _NUM_EXPS[0], _HID)


def _layer0(coords, W0, b0, g0):
    return pl.pallas_call(
        _l0_body,
        out_shape=jax.ShapeDtypeStruct((_NB * _NC, _HID), jnp.float32),
    )(coords, W0, b0, g0)


def _mid_body(x_ref, w_ref, b_ref, idx_ref, gs_ref, out_ref, *, ne, k, tile,
              dense=False):
    h = tile // ne
    u = jnp.dot(x_ref[...], w_ref[...],
                preferred_element_type=jnp.float32) + b_ref[...]
    if dense:
        y = jnp.sin(30.0 * u)
        for b in range(_NB):
            out_ref[0, pl.ds(b * _NC, _NC), :] = _combine_matmul(
                y[b * _NC:(b + 1) * _NC, :], gs_ref[b:b + 1, :], ne, h)
        return
    for b in range(_NB):
        ub = u[b * _NC:(b + 1) * _NC, :]
        i2 = jnp.broadcast_to(idx_ref[b:b + 1, :], (_NC, k))
        parts = []
        if ne == 64:
            i2p = jnp.broadcast_to(
                jnp.concatenate([idx_ref[b:b + 1, :],
                                 idx_ref[b:b + 1, :] + ne], axis=1),
                (_NC, 2 * k))
            for c in range(h // 2):
                seg = ub[:, c * 128:(c + 1) * 128]
                parts.append(jnp.take_along_axis(seg, i2p, axis=1))
        for dd in (range(h) if ne > 64 else ()):
            seg = ub[:, dd * ne:(dd + 1) * ne]
            if ne <= 128:
                parts.append(jnp.take_along_axis(seg, i2, axis=1))
            else:
                p0 = jnp.take_along_axis(seg[:, :128],
                                         jnp.minimum(i2, 127), axis=1)
                p1 = jnp.take_along_axis(seg[:, 128:],
                                         jnp.maximum(i2 - 128, 0), axis=1)
                parts.append(jnp.where(i2 < 128, p0, p1))
        ysel = jnp.sin(30.0 * jnp.concatenate(parts, axis=1))
        out_ref[0, pl.ds(b * _NC, _NC), :] = _combine_matmul(
            ysel, gs_ref[b:b + 1, :], k, h)


def _mid_layer(x, W, b, idx, gs, *, ne, k, d, tile, dense=False):
    ncols = W.shape[1]
    steps = ncols // tile
    h = tile // ne
    gs_cols = ne if dense else k
    body = functools.partial(_mid_body, ne=ne, k=k, tile=tile, dense=dense)
    out3 = pl.pallas_call(
        body,
        grid=(steps,),
        in_specs=[
            pl.BlockSpec((_NB * _NC, _HID), lambda j: (0, 0)),
            pl.BlockSpec((_HID, tile), lambda j: (0, j)),
            pl.BlockSpec((1, tile), lambda j: (0, j)),
            pl.BlockSpec((_NB, k), lambda j: (0, 0)),
            pl.BlockSpec((_NB, gs_cols), lambda j: (0, 0)),
        ],
        out_specs=pl.BlockSpec((1, _NB * _NC, h), lambda j: (j, 0, 0)),
        out_shape=jax.ShapeDtypeStruct((steps, _NB * _NC, h), jnp.float32),
    )(x, W, b, idx, gs)
    return out3.transpose(1, 0, 2).reshape(_NB * _NC, d)


def _l4_body(x_ref, w_ref, b_ref, g_ref, out_ref):
    ne = _NUM_EXPS[4]
    u = jnp.dot(x_ref[...], w_ref[...],
                preferred_element_type=jnp.float32) + b_ref[...]
    for b in range(_NB):
        out_ref[b, :, :] = _combine_matmul(
            u[b * _NC:(b + 1) * _NC, :], g_ref[b:b + 1, :], ne, _OUT)


def _layer4(x, W4, b4, g4):
    return pl.pallas_call(
        _l4_body,
        out_shape=jax.ShapeDtypeStruct((_NB, _NC, _OUT), jnp.float32),
    )(x, W4, b4, g4)


def kernel(img, coords, Wg, bg, W0, b0, W1, b1, W2, b2, W3, b3, W4, b4,
           top_k):
    x_img = img.reshape(_NB, _GIN)
    topk_s = jnp.asarray(top_k, jnp.int32).reshape(1, 1)
    gates, i1, s1, i2, s2, i3, s3 = _compute_gates(
        x_img, Wg, bg.reshape(1, _GOUT), topk_s)

    g0 = gates[:, _OFFS[0]:_OFFS[1]]
    g4 = gates[:, _OFFS[4]:_OFFS[5]]

    x = _layer0(coords, W0, b0.reshape(1, -1), g0)
    g1 = gates[:, _OFFS[1]:_OFFS[2]]
    x = _mid_layer(x, W1, b1.reshape(1, -1), i1, g1,
                   ne=16, k=4, d=_HID, tile=2048, dense=True)
    x = _mid_layer(x, W2, b2.reshape(1, -1), i2, s2,
                   ne=64, k=32, d=_HID, tile=2048)
    x = _mid_layer(x, W3, b3.reshape(1, -1), i3, s3,
                   ne=256, k=32, d=_HID, tile=2048)
    return _layer4(x, W4, b4.reshape(1, -1), g4)


# speedup vs baseline: 2.3097x; 1.0625x over previous
import functools

import jax
import jax.numpy as jnp
from jax import lax
from jax.experimental import pallas as pl
from jax.experimental.pallas import tpu as pltpu

_NUM_EXPS = [8, 16, 64, 256, 1024]
_KS = [4, 4, 32, 32, 256]
_OFFS = [0, 8, 24, 88, 344, 1368]
_HID = 256
_OUT = 3
_NB = 2
_NC = 256
_GIN = 3072
_GOUT = 1368
_GATE_TILE = 512
_GATE_STEPS = 3


def _lower_tri(ne):
    row = lax.broadcasted_iota(jnp.int32, (ne, ne), 0)
    col = lax.broadcasted_iota(jnp.int32, (ne, ne), 1)
    return (row <= col).astype(jnp.float32)


def _topk_mask(g, k):
    ne = g.shape[1]
    bits = lax.bitcast_convert_type(jnp.abs(g), jnp.int32)

    def bs_body(_, carry):
        lo, hi = carry
        mid = lo + (hi - lo + 1) // 2
        cnt = jnp.sum((bits >= mid).astype(jnp.int32), axis=1, keepdims=True)
        ok = cnt >= k
        return jnp.where(ok, mid, lo), jnp.where(ok, hi, mid - 1)

    lo0 = jnp.zeros((_NB, 1), jnp.int32)
    hi0 = jnp.full((_NB, 1), 0x7F800000, jnp.int32)
    t, _ = lax.fori_loop(0, 32, bs_body, (lo0, hi0))

    gt = bits > t
    eq = bits == t
    c1 = jnp.sum(gt.astype(jnp.int32), axis=1, keepdims=True)
    rank = jnp.dot(eq.astype(jnp.float32), _lower_tri(ne),
                   preferred_element_type=jnp.float32)
    return gt | (eq & (rank <= (k - c1).astype(jnp.float32)))


def _col_to_row(vcol, k):
    r = lax.broadcasted_iota(jnp.int32, (k, k), 0)
    c = lax.broadcasted_iota(jnp.int32, (k, k), 1)
    return jnp.sum(jnp.where(r == c, jnp.broadcast_to(vcol, (k, k)), 0.0),
                   axis=0, keepdims=True)


def _gate_body(topk_ref, x_ref, wg_ref, bg_ref,
               out_ref, i1_ref, s1_ref, i2_ref, s2_ref, i3_ref, s3_ref,
               raw_ref):
    j = pl.program_id(0)
    raw_ref[:, pl.ds(j * _GATE_TILE, _GATE_TILE)] = jnp.dot(
        x_ref[...], wg_ref[...], preferred_element_type=jnp.float32)

    @pl.when(j == _GATE_STEPS - 1)
    def _epilogue():
        use_topk = topk_ref[0, 0] != 0
        idx_refs = {1: i1_ref, 2: i2_ref, 3: i3_ref}
        gs_refs = {1: s1_ref, 2: s2_ref, 3: s3_ref}
        for li in range(5):
            off, ne, k = _OFFS[li], _NUM_EXPS[li], _KS[li]
            g = raw_ref[:, off:off + ne] + bg_ref[:, off:off + ne]
            mask = _topk_mask(g, k)
            z = jnp.where(mask, g, 0.0)
            nrm = jnp.sqrt(jnp.sum(g * g, axis=1, keepdims=True))
            gn = g / jnp.maximum(nrm, 1e-12)
            gates = jnp.where(use_topk, z, gn)
            out_ref[:, off:off + ne] = gates
            if li not in idx_refs:
                continue
            ranksel = jnp.dot(mask.astype(jnp.float32), _lower_tri(ne),
                              preferred_element_type=jnp.float32)
            srow = lax.broadcasted_iota(jnp.int32, (k, ne), 0)
            jrow = lax.broadcasted_iota(jnp.int32, (k, ne),
                                        1).astype(jnp.float32)
            for b in range(_NB):
                rt = (jnp.broadcast_to(ranksel[b:b + 1, :], (k, ne))
                      == (srow + 1).astype(jnp.float32))
                rt = rt & jnp.broadcast_to(mask[b:b + 1, :], (k, ne))
                sel_col = jnp.sum(jnp.where(rt, jrow, 0.0),
                                  axis=1, keepdims=True)
                gs_col = jnp.sum(
                    jnp.where(rt, jnp.broadcast_to(gates[b:b + 1, :],
                                                   (k, ne)), 0.0),
                    axis=1, keepdims=True)
                idx_refs[li][b:b + 1, :] = _col_to_row(sel_col,
                                                       k).astype(jnp.int32)
                gs_refs[li][b:b + 1, :] = _col_to_row(gs_col, k)


def _compute_gates(x_img, Wg, bg, topk_s):
    outs = [jax.ShapeDtypeStruct((_NB, _GOUT), jnp.float32)]
    for li in (1, 2, 3):
        k = _KS[li]
        outs.append(jax.ShapeDtypeStruct((_NB, k), jnp.int32))
        outs.append(jax.ShapeDtypeStruct((_NB, k), jnp.float32))
    full = lambda j: tuple(0 for _ in range(2))
    return pl.pallas_call(
        _gate_body,
        grid=(_GATE_STEPS,),
        in_specs=[
            pl.BlockSpec(memory_space=pltpu.SMEM),
            pl.BlockSpec((_NB, _GIN), lambda j: (0, 0)),
            pl.BlockSpec((_GIN, _GATE_TILE), lambda j: (0, j)),
            pl.BlockSpec((1, _GOUT), lambda j: (0, 0)),
        ],
        out_specs=[
            pl.BlockSpec((_NB, _GOUT), full),
            pl.BlockSpec((_NB, _KS[1]), full), pl.BlockSpec((_NB, _KS[1]), full),
            pl.BlockSpec((_NB, _KS[2]), full), pl.BlockSpec((_NB, _KS[2]), full),
            pl.BlockSpec((_NB, _KS[3]), full), pl.BlockSpec((_NB, _KS[3]), full),
        ],
        out_shape=outs,
        scratch_shapes=[pltpu.VMEM((_NB, _GATE_TILE * _GATE_STEPS),
                                   jnp.float32)],
    )(topk_s, x_img, Wg, bg)


def _gate_col(g_row, tile, ne):
    if ne <= 16:
        r = lax.broadcasted_iota(jnp.int32, (tile, 1), 0) % ne
        gcol = jnp.zeros((tile, 1), jnp.float32)
        for e in range(ne):
            gcol = jnp.where(r == e, g_row[0, e], gcol)
        return gcol
    jj = lax.broadcasted_iota(jnp.int32, (tile, ne), 0)
    ee = lax.broadcasted_iota(jnp.int32, (tile, ne), 1)
    sel = (jj % ne) == ee
    return jnp.sum(jnp.where(sel, jnp.broadcast_to(g_row, (tile, ne)), 0.0),
                   axis=1, keepdims=True)


def _combine_matmul(yb, g_row, ne, h):
    tile = h * ne
    gcol = _gate_col(g_row, tile, ne)
    jj = lax.broadcasted_iota(jnp.int32, (tile, h), 0)
    dd = lax.broadcasted_iota(jnp.int32, (tile, h), 1)
    gmat = jnp.where(jj // ne == dd, jnp.broadcast_to(gcol, (tile, h)), 0.0)
    return jnp.dot(yb, gmat, preferred_element_type=jnp.float32)


def _l0_body(coords_ref, w_ref, b_ref, g_ref, out_ref):
    y = jnp.sin(30.0 * (jnp.dot(coords_ref[...], w_ref[...],
                                preferred_element_type=jnp.float32)
                        + b_ref[...]))
    for b in range(_NB):
        out_ref[pl.ds(b * _NC, _NC), :] = _combine_matmul(
            y, g_ref[b:b + 1, :], _NUM_EXPS[0], _HID)


def _layer0(coords, W0, b0, g0):
    return pl.pallas_call(
        _l0_body,
        out_shape=jax.ShapeDtypeStruct((_NB * _NC, _HID), jnp.float32),
    )(coords, W0, b0, g0)


def _mid_body(x_ref, w_ref, b_ref, idx_ref, gs_ref, out_ref, *, ne, k, tile,
              dense=False):
    h = tile // ne
    u = jnp.dot(x_ref[...], w_ref[...],
                preferred_element_type=jnp.float32) + b_ref[...]
    if dense:
        y = jnp.sin(30.0 * u)
        for b in range(_NB):
            out_ref[0, pl.ds(b * _NC, _NC), :] = _combine_matmul(
                y[b * _NC:(b + 1) * _NC, :], gs_ref[b:b + 1, :], ne, h)
        return
    for b in range(_NB):
        ub = u[b * _NC:(b + 1) * _NC, :]
        i2 = jnp.broadcast_to(idx_ref[b:b + 1, :], (_NC, k))
        parts = []
        if ne == 64:
            i2p = jnp.broadcast_to(
                jnp.concatenate([idx_ref[b:b + 1, :],
                                 idx_ref[b:b + 1, :] + ne], axis=1),
                (_NC, 2 * k))
            for c in range(h // 2):
                seg = ub[:, c * 128:(c + 1) * 128]
                parts.append(jnp.take_along_axis(seg, i2p, axis=1))
        for dd in (range(h) if ne > 64 else ()):
            seg = ub[:, dd * ne:(dd + 1) * ne]
            if ne <= 128:
                parts.append(jnp.take_along_axis(seg, i2, axis=1))
            else:
                p0 = jnp.take_along_axis(seg[:, :128],
                                         jnp.minimum(i2, 127), axis=1)
                p1 = jnp.take_along_axis(seg[:, 128:],
                                         jnp.maximum(i2 - 128, 0), axis=1)
                parts.append(jnp.where(i2 < 128, p0, p1))
        ysel = jnp.sin(30.0 * jnp.concatenate(parts, axis=1))
        out_ref[0, pl.ds(b * _NC, _NC), :] = _combine_matmul(
            ysel, gs_ref[b:b + 1, :], k, h)


def _mid_layer(x, W, b, idx, gs, *, ne, k, d, tile, dense=False):
    ncols = W.shape[1]
    steps = ncols // tile
    h = tile // ne
    gs_cols = ne if dense else k
    body = functools.partial(_mid_body, ne=ne, k=k, tile=tile, dense=dense)
    out3 = pl.pallas_call(
        body,
        grid=(steps,),
        in_specs=[
            pl.BlockSpec((_NB * _NC, _HID), lambda j: (0, 0)),
            pl.BlockSpec((_HID, tile), lambda j: (0, j)),
            pl.BlockSpec((1, tile), lambda j: (0, j)),
            pl.BlockSpec((_NB, k), lambda j: (0, 0)),
            pl.BlockSpec((_NB, gs_cols), lambda j: (0, 0)),
        ],
        out_specs=pl.BlockSpec((1, _NB * _NC, h), lambda j: (j, 0, 0)),
        out_shape=jax.ShapeDtypeStruct((steps, _NB * _NC, h), jnp.float32),
    )(x, W, b, idx, gs)
    return out3.transpose(1, 0, 2).reshape(_NB * _NC, d)


def _l4_body(x_ref, w_ref, b_ref, g_ref, out_ref):
    ne = _NUM_EXPS[4]
    u = jnp.dot(x_ref[...], w_ref[...],
                preferred_element_type=jnp.float32) + b_ref[...]
    for b in range(_NB):
        out_ref[b, :, :] = _combine_matmul(
            u[b * _NC:(b + 1) * _NC, :], g_ref[b:b + 1, :], ne, _OUT)


def _layer4(x, W4, b4, g4):
    return pl.pallas_call(
        _l4_body,
        out_shape=jax.ShapeDtypeStruct((_NB, _NC, _OUT), jnp.float32),
    )(x, W4, b4, g4)


def kernel(img, coords, Wg, bg, W0, b0, W1, b1, W2, b2, W3, b3, W4, b4,
           top_k):
    x_img = img.reshape(_NB, _GIN)
    topk_s = jnp.asarray(top_k, jnp.int32).reshape(1, 1)
    gates, i1, s1, i2, s2, i3, s3 = _compute_gates(
        x_img, Wg, bg.reshape(1, _GOUT), topk_s)

    g0 = gates[:, _OFFS[0]:_OFFS[1]]
    g4 = gates[:, _OFFS[4]:_OFFS[5]]

    x = _layer0(coords, W0, b0.reshape(1, -1), g0)
    g1 = gates[:, _OFFS[1]:_OFFS[2]]
    x = _mid_layer(x, W1, b1.reshape(1, -1), i1, g1,
                   ne=16, k=4, d=_HID, tile=4096, dense=True)
    x = _mid_layer(x, W2, b2.reshape(1, -1), i2, s2,
                   ne=64, k=32, d=_HID, tile=2048)
    x = _mid_layer(x, W3, b3.reshape(1, -1), i3, s3,
                   ne=256, k=32, d=_HID, tile=4096)
    return _layer4(x, W4, b4.reshape(1, -1), g4)


# speedup vs baseline: 2.3772x; 1.0292x over previous
import functools

import jax
import jax.numpy as jnp
from jax import lax
from jax.experimental import pallas as pl
from jax.experimental.pallas import tpu as pltpu

_NUM_EXPS = [8, 16, 64, 256, 1024]
_KS = [4, 4, 32, 32, 256]
_OFFS = [0, 8, 24, 88, 344, 1368]
_HID = 256
_OUT = 3
_NB = 2
_NC = 256
_GIN = 3072
_GOUT = 1368
_GATE_TILE = 512
_GATE_STEPS = 3


def _lower_tri(ne):
    row = lax.broadcasted_iota(jnp.int32, (ne, ne), 0)
    col = lax.broadcasted_iota(jnp.int32, (ne, ne), 1)
    return (row <= col).astype(jnp.float32)


def _topk_mask(g, k):
    ne = g.shape[1]
    bits = lax.bitcast_convert_type(jnp.abs(g), jnp.int32)

    def bs_body(_, carry):
        lo, hi = carry
        mid = lo + (hi - lo + 1) // 2
        cnt = jnp.sum((bits >= mid).astype(jnp.int32), axis=1, keepdims=True)
        ok = cnt >= k
        return jnp.where(ok, mid, lo), jnp.where(ok, hi, mid - 1)

    lo0 = jnp.zeros((_NB, 1), jnp.int32)
    hi0 = jnp.full((_NB, 1), 0x7F800000, jnp.int32)
    t, _ = lax.fori_loop(0, 32, bs_body, (lo0, hi0))

    gt = bits > t
    eq = bits == t
    c1 = jnp.sum(gt.astype(jnp.int32), axis=1, keepdims=True)
    rank = jnp.dot(eq.astype(jnp.float32), _lower_tri(ne),
                   preferred_element_type=jnp.float32)
    return gt | (eq & (rank <= (k - c1).astype(jnp.float32)))


def _col_to_row(vcol, k):
    r = lax.broadcasted_iota(jnp.int32, (k, k), 0)
    c = lax.broadcasted_iota(jnp.int32, (k, k), 1)
    return jnp.sum(jnp.where(r == c, jnp.broadcast_to(vcol, (k, k)), 0.0),
                   axis=0, keepdims=True)


def _gate_body(topk_ref, x_ref, wg_ref, bg_ref,
               out_ref, i1_ref, s1_ref, i2_ref, s2_ref, i3_ref, s3_ref,
               raw_ref):
    j = pl.program_id(0)
    raw_ref[:, pl.ds(j * _GATE_TILE, _GATE_TILE)] = jnp.dot(
        x_ref[...], wg_ref[...], preferred_element_type=jnp.float32)

    @pl.when(j == _GATE_STEPS - 1)
    def _epilogue():
        use_topk = topk_ref[0, 0] != 0
        idx_refs = {1: i1_ref, 2: i2_ref, 3: i3_ref}
        gs_refs = {1: s1_ref, 2: s2_ref, 3: s3_ref}
        for li in range(5):
            off, ne, k = _OFFS[li], _NUM_EXPS[li], _KS[li]
            g = raw_ref[:, off:off + ne] + bg_ref[:, off:off + ne]
            mask = _topk_mask(g, k)
            z = jnp.where(mask, g, 0.0)
            nrm = jnp.sqrt(jnp.sum(g * g, axis=1, keepdims=True))
            gn = g / jnp.maximum(nrm, 1e-12)
            gates = jnp.where(use_topk, z, gn)
            out_ref[:, off:off + ne] = gates
            if li not in idx_refs:
                continue
            ranksel = jnp.dot(mask.astype(jnp.float32), _lower_tri(ne),
                              preferred_element_type=jnp.float32)
            srow = lax.broadcasted_iota(jnp.int32, (k, ne), 0)
            jrow = lax.broadcasted_iota(jnp.int32, (k, ne),
                                        1).astype(jnp.float32)
            for b in range(_NB):
                rt = (jnp.broadcast_to(ranksel[b:b + 1, :], (k, ne))
                      == (srow + 1).astype(jnp.float32))
                rt = rt & jnp.broadcast_to(mask[b:b + 1, :], (k, ne))
                sel_col = jnp.sum(jnp.where(rt, jrow, 0.0),
                                  axis=1, keepdims=True)
                gs_col = jnp.sum(
                    jnp.where(rt, jnp.broadcast_to(gates[b:b + 1, :],
                                                   (k, ne)), 0.0),
                    axis=1, keepdims=True)
                idx_refs[li][b:b + 1, :] = _col_to_row(sel_col,
                                                       k).astype(jnp.int32)
                gs_refs[li][b:b + 1, :] = _col_to_row(gs_col, k)


def _compute_gates(x_img, Wg, bg, topk_s):
    outs = [jax.ShapeDtypeStruct((_NB, _GOUT), jnp.float32)]
    for li in (1, 2, 3):
        k = _KS[li]
        outs.append(jax.ShapeDtypeStruct((_NB, k), jnp.int32))
        outs.append(jax.ShapeDtypeStruct((_NB, k), jnp.float32))
    full = lambda j: tuple(0 for _ in range(2))
    return pl.pallas_call(
        _gate_body,
        grid=(_GATE_STEPS,),
        in_specs=[
            pl.BlockSpec(memory_space=pltpu.SMEM),
            pl.BlockSpec((_NB, _GIN), lambda j: (0, 0)),
            pl.BlockSpec((_GIN, _GATE_TILE), lambda j: (0, j)),
            pl.BlockSpec((1, _GOUT), lambda j: (0, 0)),
        ],
        out_specs=[
            pl.BlockSpec((_NB, _GOUT), full),
            pl.BlockSpec((_NB, _KS[1]), full), pl.BlockSpec((_NB, _KS[1]), full),
            pl.BlockSpec((_NB, _KS[2]), full), pl.BlockSpec((_NB, _KS[2]), full),
            pl.BlockSpec((_NB, _KS[3]), full), pl.BlockSpec((_NB, _KS[3]), full),
        ],
        out_shape=outs,
        scratch_shapes=[pltpu.VMEM((_NB, _GATE_TILE * _GATE_STEPS),
                                   jnp.float32)],
    )(topk_s, x_img, Wg, bg)


def _gate_col(g_row, tile, ne):
    if ne <= 16:
        r = lax.broadcasted_iota(jnp.int32, (tile, 1), 0) % ne
        gcol = jnp.zeros((tile, 1), jnp.float32)
        for e in range(ne):
            gcol = jnp.where(r == e, g_row[0, e], gcol)
        return gcol
    jj = lax.broadcasted_iota(jnp.int32, (tile, ne), 0)
    ee = lax.broadcasted_iota(jnp.int32, (tile, ne), 1)
    sel = (jj % ne) == ee
    return jnp.sum(jnp.where(sel, jnp.broadcast_to(g_row, (tile, ne)), 0.0),
                   axis=1, keepdims=True)


def _combine_matmul(yb, g_row, ne, h):
    tile = h * ne
    gcol = _gate_col(g_row, tile, ne)
    jj = lax.broadcasted_iota(jnp.int32, (tile, h), 0)
    dd = lax.broadcasted_iota(jnp.int32, (tile, h), 1)
    gmat = jnp.where(jj // ne == dd, jnp.broadcast_to(gcol, (tile, h)), 0.0)
    return jnp.dot(yb, gmat, preferred_element_type=jnp.float32)


def _l0_body(coords_ref, w_ref, b_ref, g_ref, out_ref):
    y = jnp.sin(30.0 * (jnp.dot(coords_ref[...], w_ref[...],
                                preferred_element_type=jnp.float32)
                        + b_ref[...]))
    for b in range(_NB):
        out_ref[pl.ds(b * _NC, _NC), :] = _combine_matmul(
            y, g_ref[b:b + 1, :], _NUM_EXPS[0], _HID)


def _layer0(coords, W0, b0, g0):
    return pl.pallas_call(
        _l0_body,
        out_shape=jax.ShapeDtypeStruct((_NB * _NC, _HID), jnp.float32),
    )(coords, W0, b0, g0)


def _mid_body(x_ref, w_ref, b_ref, idx_ref, gs_ref, out_ref, *, ne, k, tile,
              dense=False):
    h = tile // ne
    u = jnp.dot(x_ref[...], w_ref[...],
                preferred_element_type=jnp.float32) + b_ref[...]
    if dense:
        y = jnp.sin(30.0 * u)
        for b in range(_NB):
            out_ref[0, pl.ds(b * _NC, _NC), :] = _combine_matmul(
                y[b * _NC:(b + 1) * _NC, :], gs_ref[b:b + 1, :], ne, h)
        return
    for b in range(_NB):
        ub = u[b * _NC:(b + 1) * _NC, :]
        i2 = jnp.broadcast_to(idx_ref[b:b + 1, :], (_NC, k))
        parts = []
        if ne == 64:
            i2p = jnp.broadcast_to(
                jnp.concatenate([idx_ref[b:b + 1, :],
                                 idx_ref[b:b + 1, :] + ne], axis=1),
                (_NC, 2 * k))
            for c in range(h // 2):
                seg = ub[:, c * 128:(c + 1) * 128]
                parts.append(jnp.take_along_axis(seg, i2p, axis=1))
        for dd in (range(h) if ne > 64 else ()):
            seg = ub[:, dd * ne:(dd + 1) * ne]
            if ne <= 128:
                parts.append(jnp.take_along_axis(seg, i2, axis=1))
            else:
                p0 = jnp.take_along_axis(seg[:, :128],
                                         jnp.minimum(i2, 127), axis=1)
                p1 = jnp.take_along_axis(seg[:, 128:],
                                         jnp.maximum(i2 - 128, 0), axis=1)
                parts.append(jnp.where(i2 < 128, p0, p1))
        ysel = jnp.sin(30.0 * jnp.concatenate(parts, axis=1))
        out_ref[0, pl.ds(b * _NC, _NC), :] = _combine_matmul(
            ysel, gs_ref[b:b + 1, :], k, h)


def _mid_layer(x, W, b, idx, gs, *, ne, k, d, tile, dense=False):
    ncols = W.shape[1]
    steps = ncols // tile
    h = tile // ne
    gs_cols = ne if dense else k
    body = functools.partial(_mid_body, ne=ne, k=k, tile=tile, dense=dense)
    out3 = pl.pallas_call(
        body,
        grid=(steps,),
        in_specs=[
            pl.BlockSpec((_NB * _NC, _HID), lambda j: (0, 0)),
            pl.BlockSpec((_HID, tile), lambda j: (0, j)),
            pl.BlockSpec((1, tile), lambda j: (0, j)),
            pl.BlockSpec((_NB, k), lambda j: (0, 0)),
            pl.BlockSpec((_NB, gs_cols), lambda j: (0, 0)),
        ],
        out_specs=pl.BlockSpec((1, _NB * _NC, h), lambda j: (j, 0, 0)),
        out_shape=jax.ShapeDtypeStruct((steps, _NB * _NC, h), jnp.float32),
    )(x, W, b, idx, gs)
    return out3.transpose(1, 0, 2).reshape(_NB * _NC, d)


def _l4_body(x_ref, w_ref, b_ref, g_ref, out_ref):
    ne = _NUM_EXPS[4]
    u = jnp.dot(x_ref[...], w_ref[...],
                preferred_element_type=jnp.float32) + b_ref[...]
    for b in range(_NB):
        out_ref[b, :, :] = _combine_matmul(
            u[b * _NC:(b + 1) * _NC, :], g_ref[b:b + 1, :], ne, _OUT)


def _layer4(x, W4, b4, g4):
    return pl.pallas_call(
        _l4_body,
        out_shape=jax.ShapeDtypeStruct((_NB, _NC, _OUT), jnp.float32),
    )(x, W4, b4, g4)


def kernel(img, coords, Wg, bg, W0, b0, W1, b1, W2, b2, W3, b3, W4, b4,
           top_k):
    x_img = img.reshape(_NB, _GIN)
    topk_s = jnp.asarray(top_k, jnp.int32).reshape(1, 1)
    gates, i1, s1, i2, s2, i3, s3 = _compute_gates(
        x_img, Wg, bg.reshape(1, _GOUT), topk_s)

    g0 = gates[:, _OFFS[0]:_OFFS[1]]
    g4 = gates[:, _OFFS[4]:_OFFS[5]]

    x = _layer0(coords, W0, b0.reshape(1, -1), g0)
    g1 = gates[:, _OFFS[1]:_OFFS[2]]
    x = _mid_layer(x, W1, b1.reshape(1, -1), i1, g1,
                   ne=16, k=4, d=_HID, tile=4096, dense=True)
    x = _mid_layer(x, W2, b2.reshape(1, -1), i2, s2,
                   ne=64, k=32, d=_HID, tile=4096)
    x = _mid_layer(x, W3, b3.reshape(1, -1), i3, s3,
                   ne=256, k=32, d=_HID, tile=4096)
    return _layer4(x, W4, b4.reshape(1, -1), g4)
